# baseline (speedup 1.0000x reference)
import dataclasses
import functools

import jax
import jax.numpy as jnp
from jax import lax
from jax.experimental import pallas as pl
from jax.experimental.pallas import tpu as pltpu
from jax.experimental.pallas import tpu_sc as plsc

K = 50
NUM_ANSWERS = 1000
Q = 1024
D = 128
N = 100000
BN = 2048
NPAD = 100352


CH = 128
NCH = NPAD // CH
KSEL = 51


def _mm_body(q_ref, kt_ref, o_ref):
    i = pl.program_id(0)
    s = jax.lax.dot_general(
        q_ref[...], kt_ref[...], (((1,), (0,)), ((), ())),
        preferred_element_type=jnp.float32,
    )
    col = i * BN + jax.lax.broadcasted_iota(jnp.int32, s.shape, 1)
    o_ref[...] = jnp.where(col < N, s, 3.0)


def _sims(qn, knt):
    return pl.pallas_call(
        _mm_body,
        grid=(NPAD // BN,),
        in_specs=[
            pl.BlockSpec((Q, D), lambda i: (0, 0)),
            pl.BlockSpec((D, BN), lambda i: (0, i)),
        ],
        out_specs=pl.BlockSpec((Q, BN), lambda i: (0, i)),
        out_shape=jax.ShapeDtypeStruct((Q, NPAD), jnp.float32),
    )(qn, knt)


def _mmT_body(k_ref, qt_ref, m_ref):
    i = pl.program_id(0)
    s = jax.lax.dot_general(
        k_ref[...], qt_ref[...], (((1,), (0,)), ((), ())),
        preferred_element_type=jnp.float32,
    )
    row = i * BN + jax.lax.broadcasted_iota(jnp.int32, s.shape, 0)
    s = jnp.where(row < N, s, 3.0)
    m_ref[...] = jnp.min(s.reshape(BN // CH, CH, Q), axis=1)


def _segminT(knp, qnt):
    return pl.pallas_call(
        _mmT_body,
        grid=(NPAD // BN,),
        in_specs=[
            pl.BlockSpec((BN, D), lambda i: (i, 0)),
            pl.BlockSpec((D, Q), lambda i: (0, 0)),
        ],
        out_specs=pl.BlockSpec((BN // CH, Q), lambda i: (i, 0)),
        out_shape=jax.ShapeDtypeStruct((NCH, Q), jnp.float32),
    )(knp, qnt)


def _bisect_body(m_ref, o_ref):
    seg = m_ref[...]

    def step(_, lohi):
        lo, hi = lohi
        mid = (lo + hi) * 0.5
        cnt = jnp.sum((seg <= mid).astype(jnp.float32), axis=0, keepdims=True)
        ge = cnt >= KSEL
        return jnp.where(ge, lo, mid), jnp.where(ge, mid, hi)

    lo0 = jnp.full((1, Q), -1.5, jnp.float32)
    hi0 = jnp.full((1, Q), 3.5, jnp.float32)
    _, hi = jax.lax.fori_loop(0, 40, step, (lo0, hi0))
    o_ref[...] = hi


def _v51(segmint):
    return pl.pallas_call(
        _bisect_body,
        out_shape=jax.ShapeDtypeStruct((1, Q), jnp.float32),
    )(segmint)


NW = 16
QPW = Q // NW
L = 16
CAPCH = 64
IDBUF = CAPCH + L
CANDB = CAPCH * CH + L
BIGF = 1e30
BIGI = 2**31 - 1
PADLAB = 1 << 20


def _sc_body(sim2_hbm, seg_hbm, v51_hbm, ans2_hbm, out_hbm,
             segb, v51b, idsb, rowb, gb, ab, cval, clab,
             kbuf, cbuf, outb):
    wid = lax.axis_index("s")
    qbase = wid * QPW
    pltpu.sync_copy(v51_hbm.at[pl.ds(qbase, QPW)], v51b)
    lane = lax.iota(jnp.int32, L)

    for j in range(QPW // L):
        def qstep(t, outv):
            q = qbase + j * L + t
            vv = v51b[pl.ds(j * L, L)]
            tsc = lax.reduce_min(jnp.where(lane == t, vv, BIGF), (0,))
            tv = jnp.full((L,), tsc)
            pltpu.sync_copy(seg_hbm.at[q], segb)

            for g in range(IDBUF // L):
                idsb[pl.ds(g * L, L)] = jnp.zeros((L,), jnp.int32)

            def p1(g, off):
                m = segb[pl.ds(g * L, L)] <= tv
                m = jnp.logical_and(m, jnp.full((L,), off < CAPCH))
                plsc.store_compressed(idsb.at[pl.ds(off, L)], g * L + lane, mask=m)
                return off + lax.reduce_max(
                    plsc.all_reduce_population_count(m), (0,))

            off = lax.fori_loop(0, NCH // L, p1, jnp.int32(0))
            nch = jnp.minimum(off, jnp.int32(CAPCH))

            qrow = jnp.full((L,), q * NCH, jnp.int32)
            for g in range(CAPCH // L):
                rowb[pl.ds(g * L, L)] = idsb[pl.ds(g * L, L)] + qrow

            pltpu.sync_copy(sim2_hbm.at[rowb.at[pl.ds(0, CAPCH)]], gb)
            pltpu.sync_copy(ans2_hbm.at[idsb.at[pl.ds(0, CAPCH)]], ab)

            def p2(r, offc):
                res = offc
                for c in range(0, CH, L):
                    v = gb[r, pl.ds(c, L)]
                    m = v <= tv
                    plsc.store_compressed(cval.at[pl.ds(res, L)], v, mask=m)
                    plsc.store_compressed(clab.at[pl.ds(res, L)],
                                          ab[r, pl.ds(c, L)], mask=m)
                    res = res + lax.reduce_max(
                        plsc.all_reduce_population_count(m), (0,))
                return res

            ccnt = lax.fori_loop(0, nch, p2, jnp.int32(0))
            cval[pl.ds(ccnt, L)] = jnp.full((L,), BIGF, jnp.float32)
            clab[pl.ds(ccnt, L)] = jnp.full((L,), PADLAB, jnp.int32)
            nv = (ccnt + L - 1) // L

            def p3(ga, _):
                va = cval[pl.ds(ga * L, L)]
                pa = ga * L + lane

                def p3b(gb_, rank):
                    vb = cval[pl.ds(gb_ * L, L)]
                    for k in range(L):
                        ridx = (lane + k) & (L - 1)
                        vr = vb.at[ridx].get(mode="promise_in_bounds")
                        pr = gb_ * L + ridx
                        less = (vr < va) | ((vr == va) & (pr < pa))
                        rank = rank + less.astype(jnp.int32)
                    return rank

                rank = lax.fori_loop(0, nv, p3b, jnp.zeros((L,), jnp.int32))
                kbuf[pl.ds(ga * L, L)] = (rank < K).astype(jnp.int32)
                return _

            lax.fori_loop(0, nv, p3, jnp.int32(0))

            def p4(ga, mc):
                la = clab[pl.ds(ga * L, L)]
                ka = kbuf[pl.ds(ga * L, L)]

                def p4b(gb_, cnt):
                    lb = clab[pl.ds(gb_ * L, L)]
                    kb = kbuf[pl.ds(gb_ * L, L)]
                    for k in range(L):
                        ridx = (lane + k) & (L - 1)
                        lr = lb.at[ridx].get(mode="promise_in_bounds")
                        kr = kb.at[ridx].get(mode="promise_in_bounds")
                        cnt = cnt + jnp.where(lr == la, kr, 0)
                    return cnt

                cnt = lax.fori_loop(0, nv, p4b, jnp.zeros((L,), jnp.int32))
                cbuf[pl.ds(ga * L, L)] = cnt
                return jnp.maximum(mc, lax.reduce_max(
                    jnp.where(ka > 0, cnt, -1), (0,)))

            maxc = lax.fori_loop(0, nv, p4, jnp.int32(-1))

            def p5(ga, w):
                la = clab[pl.ds(ga * L, L)]
                ka = kbuf[pl.ds(ga * L, L)]
                ca = cbuf[pl.ds(ga * L, L)]
                sel = (ka > 0) & (ca == maxc)
                return jnp.minimum(w, lax.reduce_min(
                    jnp.where(sel, la, BIGI), (0,)))

            win = lax.fori_loop(0, nv, p5, jnp.int32(BIGI))
            return jnp.where(lane == t, win, outv)

        outv = lax.fori_loop(0, L, qstep, jnp.zeros((L,), jnp.int32))
        outb[pl.ds(j * L, L)] = outv

    pltpu.sync_copy(outb, out_hbm.at[pl.ds(qbase, QPW)])


def _sc_params():
    cp = pltpu.CompilerParams()
    if "needs_layout_passes" in pltpu.CompilerParams.__dataclass_fields__:
        cp = dataclasses.replace(cp, needs_layout_passes=False)
    return cp


def _sc_select(sim2, segmin, v51, ans2):
    fn = pl.kernel(
        _sc_body,
        mesh=plsc.VectorSubcoreMesh(core_axis_name="c", subcore_axis_name="s", num_cores=1),
        compiler_params=_sc_params(),
        out_type=jax.ShapeDtypeStruct((Q,), jnp.int32),
        scratch_types=[
            pltpu.VMEM((NCH,), jnp.float32),
            pltpu.VMEM((QPW,), jnp.float32),
            pltpu.VMEM((IDBUF,), jnp.int32),
            pltpu.VMEM((IDBUF,), jnp.int32),
            pltpu.VMEM((CAPCH, CH), jnp.float32),
            pltpu.VMEM((CAPCH, CH), jnp.int32),
            pltpu.VMEM((CANDB,), jnp.float32),
            pltpu.VMEM((CANDB,), jnp.int32),
            pltpu.VMEM((CANDB,), jnp.int32),
            pltpu.VMEM((CANDB,), jnp.int32),
            pltpu.VMEM((QPW,), jnp.int32),
        ],
    )
    return fn(sim2, segmin, v51, ans2)


def kernel(queries, keys, answers):
    qn = queries / (jnp.linalg.norm(queries, axis=1, keepdims=True) + 1e-8)
    kn = keys / (jnp.linalg.norm(keys, axis=1, keepdims=True) + 1e-8)
    knp = jnp.pad(kn, ((0, NPAD - N), (0, 0)))
    sims = _sims(qn, knp.T)
    segmint = _segminT(knp, qn.T)
    segmin = segmint.T
    v51 = _v51(segmint).reshape(Q)
    sim2 = sims.reshape(Q * NCH, CH)
    ans2 = jnp.pad(answers.astype(jnp.int32), (0, NPAD - N)).reshape(NCH, CH)
    return _sc_select(sim2, segmin, v51, ans2)


# speedup vs baseline: 1.2108x; 1.2108x over previous
import dataclasses

import jax
import jax.numpy as jnp
from jax import lax
from jax.experimental import pallas as pl
from jax.experimental.pallas import tpu as pltpu
from jax.experimental.pallas import tpu_sc as plsc

K = 50
NUM_ANSWERS = 1000
Q = 1024
D = 128
N = 100000
BN = 2048
NPAD = 100352

CH = 128
NCH = NPAD // CH
KSEL = 51


def _mm_body(q_ref, kt_ref, o_ref):
    i = pl.program_id(0)
    s = jax.lax.dot_general(
        q_ref[...], kt_ref[...], (((1,), (0,)), ((), ())),
        preferred_element_type=jnp.float32,
    )
    col = i * BN + jax.lax.broadcasted_iota(jnp.int32, s.shape, 1)
    o_ref[...] = jnp.where(col < N, s, 3.0)


def _sims(qn, knt):
    return pl.pallas_call(
        _mm_body,
        grid=(NPAD // BN,),
        in_specs=[
            pl.BlockSpec((Q, D), lambda i: (0, 0)),
            pl.BlockSpec((D, BN), lambda i: (0, i)),
        ],
        out_specs=pl.BlockSpec((Q, BN), lambda i: (0, i)),
        out_shape=jax.ShapeDtypeStruct((Q, NPAD), jnp.float32),
    )(qn, knt)


def _mmT_body(k_ref, qt_ref, m_ref):
    i = pl.program_id(0)
    s = jax.lax.dot_general(
        k_ref[...], qt_ref[...], (((1,), (0,)), ((), ())),
        preferred_element_type=jnp.float32,
    )
    row = i * BN + jax.lax.broadcasted_iota(jnp.int32, s.shape, 0)
    s = jnp.where(row < N, s, 3.0)
    m_ref[...] = jnp.min(s.reshape(BN // CH, CH, Q), axis=1)


def _segminT(knp, qnt):
    return pl.pallas_call(
        _mmT_body,
        grid=(NPAD // BN,),
        in_specs=[
            pl.BlockSpec((BN, D), lambda i: (i, 0)),
            pl.BlockSpec((D, Q), lambda i: (0, 0)),
        ],
        out_specs=pl.BlockSpec((BN // CH, Q), lambda i: (i, 0)),
        out_shape=jax.ShapeDtypeStruct((NCH, Q), jnp.float32),
    )(knp, qnt)


def _bisect_body(m_ref, o_ref):
    seg = m_ref[...]

    def step(_, lohi):
        lo, hi = lohi
        mid = (lo + hi) * 0.5
        cnt = jnp.sum((seg <= mid).astype(jnp.float32), axis=0, keepdims=True)
        ge = cnt >= KSEL
        return jnp.where(ge, lo, mid), jnp.where(ge, mid, hi)

    lo0 = jnp.full((1, Q), -1.5, jnp.float32)
    hi0 = jnp.full((1, Q), 3.5, jnp.float32)
    _, hi = jax.lax.fori_loop(0, 40, step, (lo0, hi0))
    o_ref[...] = hi


def _v51(segmint):
    return pl.pallas_call(
        _bisect_body,
        out_shape=jax.ShapeDtypeStruct((1, Q), jnp.float32),
    )(segmint)


NW = 32
QPW = Q // NW
L = 16
CAPCH = 64
IDBUF = CAPCH + L
CANDB = CAPCH * CH + L
BIGF = 1e30
BIGI = 2**31 - 1
PADLAB = 1 << 20


def _sc_body(sim2_hbm, seg_hbm, v51_hbm, ans2_hbm, out_hbm,
             segb, v51b, idsb, rowb, gb, ab, cval, clab,
             kbuf, cbuf, outb):
    wid = lax.axis_index("s") * 2 + lax.axis_index("c")
    qbase = wid * QPW
    pltpu.sync_copy(v51_hbm.at[pl.ds(qbase, QPW)], v51b)
    lane = lax.iota(jnp.int32, L)

    for j in range(QPW // L):
        def qstep(t, outv):
            q = qbase + j * L + t
            vv = v51b[pl.ds(j * L, L)]
            tsc = lax.reduce_min(jnp.where(lane == t, vv, BIGF), (0,))
            tv = jnp.full((L,), tsc)
            pltpu.sync_copy(seg_hbm.at[q], segb)

            for g in range(IDBUF // L):
                idsb[pl.ds(g * L, L)] = jnp.zeros((L,), jnp.int32)

            def p1(g, off):
                m = segb[pl.ds(g * L, L)] <= tv
                m = jnp.logical_and(m, jnp.full((L,), off < CAPCH))
                plsc.store_compressed(idsb.at[pl.ds(off, L)], g * L + lane,
                                      mask=m)
                return off + lax.reduce_max(
                    plsc.all_reduce_population_count(m), (0,))

            off = lax.fori_loop(0, NCH // L, p1, jnp.int32(0))
            nch = jnp.minimum(off, jnp.int32(CAPCH))

            qrow = jnp.full((L,), q * NCH, jnp.int32)
            for g in range(CAPCH // L):
                rowb[pl.ds(g * L, L)] = idsb[pl.ds(g * L, L)] + qrow

            pltpu.sync_copy(sim2_hbm.at[rowb.at[pl.ds(0, CAPCH)]], gb)
            pltpu.sync_copy(ans2_hbm.at[idsb.at[pl.ds(0, CAPCH)]], ab)

            def p2(r, offc):
                res = offc
                for c in range(0, CH, L):
                    v = gb[r, pl.ds(c, L)]
                    m = v <= tv
                    plsc.store_compressed(cval.at[pl.ds(res, L)], v, mask=m)
                    plsc.store_compressed(clab.at[pl.ds(res, L)],
                                          ab[r, pl.ds(c, L)], mask=m)
                    res = res + lax.reduce_max(
                        plsc.all_reduce_population_count(m), (0,))
                return res

            ccnt = lax.fori_loop(0, nch, p2, jnp.int32(0))
            cval[pl.ds(ccnt, L)] = jnp.full((L,), BIGF, jnp.float32)
            clab[pl.ds(ccnt, L)] = jnp.full((L,), PADLAB, jnp.int32)
            nv = (ccnt + L - 1) // L

            def p3(ga, acc):
                va = cval[pl.ds(ga * L, L)]
                pa = ga * L + lane

                def p3b(gb_, rank):
                    vb = cval[pl.ds(gb_ * L, L)]
                    for k in range(L):
                        ridx = (lane + k) & (L - 1)
                        vr = vb.at[ridx].get(mode="promise_in_bounds")
                        pr = gb_ * L + ridx
                        less = (vr < va) | ((vr == va) & (pr < pa))
                        rank = rank + less.astype(jnp.int32)
                    return rank

                rank = lax.fori_loop(0, nv, p3b, jnp.zeros((L,), jnp.int32))
                kbuf[pl.ds(ga * L, L)] = (rank < K).astype(jnp.int32)
                return acc

            lax.fori_loop(0, nv, p3, jnp.int32(0))

            def p4(ga, mc):
                la = clab[pl.ds(ga * L, L)]
                ka = kbuf[pl.ds(ga * L, L)]

                def p4b(gb_, cnt):
                    lb = clab[pl.ds(gb_ * L, L)]
                    kb = kbuf[pl.ds(gb_ * L, L)]
                    for k in range(L):
                        ridx = (lane + k) & (L - 1)
                        lr = lb.at[ridx].get(mode="promise_in_bounds")
                        kr = kb.at[ridx].get(mode="promise_in_bounds")
                        cnt = cnt + jnp.where(lr == la, kr, 0)
                    return cnt

                cnt = lax.fori_loop(0, nv, p4b, jnp.zeros((L,), jnp.int32))
                cbuf[pl.ds(ga * L, L)] = cnt
                return jnp.maximum(mc, lax.reduce_max(
                    jnp.where(ka > 0, cnt, -1), (0,)))

            maxc = lax.fori_loop(0, nv, p4, jnp.int32(-1))

            def p5(ga, w):
                la = clab[pl.ds(ga * L, L)]
                ka = kbuf[pl.ds(ga * L, L)]
                ca = cbuf[pl.ds(ga * L, L)]
                sel = (ka > 0) & (ca == maxc)
                return jnp.minimum(w, lax.reduce_min(
                    jnp.where(sel, la, BIGI), (0,)))

            win = lax.fori_loop(0, nv, p5, jnp.int32(BIGI))
            return jnp.where(lane == t, win, outv)

        outv = lax.fori_loop(0, L, qstep, jnp.zeros((L,), jnp.int32))
        outb[pl.ds(j * L, L)] = outv

    pltpu.sync_copy(outb, out_hbm.at[pl.ds(qbase, QPW)])


def _sc_params():
    cp = pltpu.CompilerParams()
    if "needs_layout_passes" in pltpu.CompilerParams.__dataclass_fields__:
        cp = dataclasses.replace(cp, needs_layout_passes=False)
    return cp


def _sc_select(sim2, segmin, v51, ans2):
    fn = pl.kernel(
        _sc_body,
        mesh=plsc.VectorSubcoreMesh(core_axis_name="c", subcore_axis_name="s"),
        compiler_params=_sc_params(),
        out_type=jax.ShapeDtypeStruct((Q,), jnp.int32),
        scratch_types=[
            pltpu.VMEM((NCH,), jnp.float32),
            pltpu.VMEM((QPW,), jnp.float32),
            pltpu.VMEM((IDBUF,), jnp.int32),
            pltpu.VMEM((IDBUF,), jnp.int32),
            pltpu.VMEM((CAPCH, CH), jnp.float32),
            pltpu.VMEM((CAPCH, CH), jnp.int32),
            pltpu.VMEM((CANDB,), jnp.float32),
            pltpu.VMEM((CANDB,), jnp.int32),
            pltpu.VMEM((CANDB,), jnp.int32),
            pltpu.VMEM((CANDB,), jnp.int32),
            pltpu.VMEM((QPW,), jnp.int32),
        ],
    )
    return fn(sim2, segmin, v51, ans2)


def kernel(queries, keys, answers):
    qn = queries / (jnp.linalg.norm(queries, axis=1, keepdims=True) + 1e-8)
    kn = keys / (jnp.linalg.norm(keys, axis=1, keepdims=True) + 1e-8)
    knp = jnp.pad(kn, ((0, NPAD - N), (0, 0)))
    sims = _sims(qn, knp.T)
    segmint = _segminT(knp, qn.T)
    segmin = segmint.T
    v51 = _v51(segmint).reshape(Q)
    sim2 = sims.reshape(Q * NCH, CH)
    ans2 = jnp.pad(answers.astype(jnp.int32), (0, NPAD - N)).reshape(NCH, CH)
    return _sc_select(sim2, segmin, v51, ans2)


# speedup vs baseline: 1.2447x; 1.0280x over previous
import dataclasses

import jax
import jax.numpy as jnp
from jax import lax
from jax.experimental import pallas as pl
from jax.experimental.pallas import tpu as pltpu
from jax.experimental.pallas import tpu_sc as plsc

K = 50
NUM_ANSWERS = 1000
Q = 1024
D = 128
N = 100000
BN = 2048
NPAD = 100352

CH = 128
NCH = NPAD // CH
KSEL = 51


def _mm_body(q_ref, kt_ref, o_ref, m_ref):
    i = pl.program_id(0)
    s = jax.lax.dot_general(
        q_ref[...], kt_ref[...], (((1,), (0,)), ((), ())),
        preferred_element_type=jnp.float32,
    )
    col = i * BN + jax.lax.broadcasted_iota(jnp.int32, s.shape, 1)
    s = jnp.where(col < N, s, 3.0)
    o_ref[...] = s
    m_ref[...] = jnp.min(s.reshape(Q, BN // CH, CH), axis=2)[None]


def _sims(qn, knt):
    return pl.pallas_call(
        _mm_body,
        grid=(NPAD // BN,),
        in_specs=[
            pl.BlockSpec((Q, D), lambda i: (0, 0)),
            pl.BlockSpec((D, BN), lambda i: (0, i)),
        ],
        out_specs=[
            pl.BlockSpec((Q, BN), lambda i: (0, i)),
            pl.BlockSpec((1, Q, BN // CH), lambda i: (i, 0, 0)),
        ],
        out_shape=[
            jax.ShapeDtypeStruct((Q, NPAD), jnp.float32),
            jax.ShapeDtypeStruct((NPAD // BN, Q, BN // CH), jnp.float32),
        ],
    )(qn, knt)


def _bisect_body(m_ref, o_ref):
    seg = m_ref[...]

    def step(_, lohi):
        lo, hi = lohi
        mid = (lo + hi) * 0.5
        cnt = jnp.sum((seg <= mid).astype(jnp.float32), axis=1, keepdims=True)
        ge = cnt >= KSEL
        return jnp.where(ge, lo, mid), jnp.where(ge, mid, hi)

    lo0 = jnp.full((Q, 1), -1.5, jnp.float32)
    hi0 = jnp.full((Q, 1), 3.5, jnp.float32)
    _, hi = jax.lax.fori_loop(0, 40, step, (lo0, hi0))
    o_ref[...] = hi


def _v51(segmin):
    return pl.pallas_call(
        _bisect_body,
        out_shape=jax.ShapeDtypeStruct((Q, 1), jnp.float32),
    )(segmin)


NW = 32
QPW = Q // NW
L = 16
CAPCH = 64
IDBUF = CAPCH + L
CANDB = CAPCH * CH + L
BIGF = 1e30
BIGI = 2**31 - 1
PADLAB = 1 << 20


def _sc_body(sim2_hbm, seg_hbm, v51_hbm, ans2_hbm, out_hbm,
             segb, v51b, idsb, rowb, gb, ab, cval, clab,
             kbuf, cbuf, outb, sem):
    wid = lax.axis_index("s") * 2 + lax.axis_index("c")
    qbase = wid * QPW
    pltpu.sync_copy(v51_hbm.at[pl.ds(qbase, QPW)], v51b)
    lane = lax.iota(jnp.int32, L)

    for j in range(QPW // L):
        def qstep(t, outv):
            q = qbase + j * L + t
            vv = v51b[pl.ds(j * L, L)]
            tsc = lax.reduce_min(jnp.where(lane == t, vv, BIGF), (0,))
            tv = jnp.full((L,), tsc)
            pltpu.sync_copy(seg_hbm.at[q], segb)

            for g in range(IDBUF // L):
                idsb[pl.ds(g * L, L)] = jnp.zeros((L,), jnp.int32)

            def p1(g, off):
                m = segb[pl.ds(g * L, L)] <= tv
                m = jnp.logical_and(m, jnp.full((L,), off < CAPCH))
                plsc.store_compressed(idsb.at[pl.ds(off, L)], g * L + lane,
                                      mask=m)
                return off + lax.reduce_max(
                    plsc.all_reduce_population_count(m), (0,))

            off = lax.fori_loop(0, NCH // L, p1, jnp.int32(0))
            nch = jnp.minimum(off, jnp.int32(CAPCH))

            qrow = jnp.full((L,), q * NCH, jnp.int32)
            for g in range(CAPCH // L):
                rowb[pl.ds(g * L, L)] = idsb[pl.ds(g * L, L)] + qrow

            cp1 = pltpu.async_copy(
                sim2_hbm.at[rowb.at[pl.ds(0, CAPCH)]], gb, sem)
            cp2 = pltpu.async_copy(
                ans2_hbm.at[idsb.at[pl.ds(0, CAPCH)]], ab, sem)
            cp1.wait()
            cp2.wait()

            def p2(r, offc):
                res = offc
                for c in range(0, CH, L):
                    v = gb[r, pl.ds(c, L)]
                    m = v <= tv
                    plsc.store_compressed(cval.at[pl.ds(res, L)], v, mask=m)
                    plsc.store_compressed(clab.at[pl.ds(res, L)],
                                          ab[r, pl.ds(c, L)], mask=m)
                    res = res + lax.reduce_max(
                        plsc.all_reduce_population_count(m), (0,))
                return res

            ccnt = lax.fori_loop(0, nch, p2, jnp.int32(0))
            cval[pl.ds(ccnt, L)] = jnp.full((L,), BIGF, jnp.float32)
            clab[pl.ds(ccnt, L)] = jnp.full((L,), PADLAB, jnp.int32)
            nv = (ccnt + L - 1) // L

            def p3(ga, acc):
                va = cval[pl.ds(ga * L, L)]
                pa = ga * L + lane

                def p3b(gb_, rank):
                    vb = cval[pl.ds(gb_ * L, L)]
                    for k in range(L):
                        ridx = (lane + k) & (L - 1)
                        vr = vb.at[ridx].get(mode="promise_in_bounds")
                        pr = gb_ * L + ridx
                        less = (vr < va) | ((vr == va) & (pr < pa))
                        rank = rank + less.astype(jnp.int32)
                    return rank

                rank = lax.fori_loop(0, nv, p3b, jnp.zeros((L,), jnp.int32))
                kbuf[pl.ds(ga * L, L)] = (rank < K).astype(jnp.int32)
                return acc

            lax.fori_loop(0, nv, p3, jnp.int32(0))

            def p4(ga, mc):
                la = clab[pl.ds(ga * L, L)]
                ka = kbuf[pl.ds(ga * L, L)]

                def p4b(gb_, cnt):
                    lb = clab[pl.ds(gb_ * L, L)]
                    kb = kbuf[pl.ds(gb_ * L, L)]
                    for k in range(L):
                        ridx = (lane + k) & (L - 1)
                        lr = lb.at[ridx].get(mode="promise_in_bounds")
                        kr = kb.at[ridx].get(mode="promise_in_bounds")
                        cnt = cnt + jnp.where(lr == la, kr, 0)
                    return cnt

                cnt = lax.fori_loop(0, nv, p4b, jnp.zeros((L,), jnp.int32))
                cbuf[pl.ds(ga * L, L)] = cnt
                return jnp.maximum(mc, lax.reduce_max(
                    jnp.where(ka > 0, cnt, -1), (0,)))

            maxc = lax.fori_loop(0, nv, p4, jnp.int32(-1))

            def p5(ga, w):
                la = clab[pl.ds(ga * L, L)]
                ka = kbuf[pl.ds(ga * L, L)]
                ca = cbuf[pl.ds(ga * L, L)]
                sel = (ka > 0) & (ca == maxc)
                return jnp.minimum(w, lax.reduce_min(
                    jnp.where(sel, la, BIGI), (0,)))

            win = lax.fori_loop(0, nv, p5, jnp.int32(BIGI))
            return jnp.where(lane == t, win, outv)

        outv = lax.fori_loop(0, L, qstep, jnp.zeros((L,), jnp.int32))
        outb[pl.ds(j * L, L)] = outv

    pltpu.sync_copy(outb, out_hbm.at[pl.ds(qbase, QPW)])


def _sc_params():
    cp = pltpu.CompilerParams()
    if "needs_layout_passes" in pltpu.CompilerParams.__dataclass_fields__:
        cp = dataclasses.replace(cp, needs_layout_passes=False)
    return cp


def _sc_select(sim2, segmin, v51, ans2):
    fn = pl.kernel(
        _sc_body,
        mesh=plsc.VectorSubcoreMesh(core_axis_name="c", subcore_axis_name="s"),
        compiler_params=_sc_params(),
        out_type=jax.ShapeDtypeStruct((Q,), jnp.int32),
        scratch_types=[
            pltpu.VMEM((NCH,), jnp.float32),
            pltpu.VMEM((QPW,), jnp.float32),
            pltpu.VMEM((IDBUF,), jnp.int32),
            pltpu.VMEM((IDBUF,), jnp.int32),
            pltpu.VMEM((CAPCH, CH), jnp.float32),
            pltpu.VMEM((CAPCH, CH), jnp.int32),
            pltpu.VMEM((CANDB,), jnp.float32),
            pltpu.VMEM((CANDB,), jnp.int32),
            pltpu.VMEM((CANDB,), jnp.int32),
            pltpu.VMEM((CANDB,), jnp.int32),
            pltpu.VMEM((QPW,), jnp.int32),
            pltpu.SemaphoreType.DMA,
        ],
    )
    return fn(sim2, segmin, v51, ans2)


def kernel(queries, keys, answers):
    qn = queries / (jnp.linalg.norm(queries, axis=1, keepdims=True) + 1e-8)
    kn = keys / (jnp.linalg.norm(keys, axis=1, keepdims=True) + 1e-8)
    knp = jnp.pad(kn, ((0, NPAD - N), (0, 0)))
    sims, segmin3 = _sims(qn, knp.T)
    segmin = segmin3.transpose(1, 0, 2).reshape(Q, NCH)
    v51 = _v51(segmin).reshape(Q)
    sim2 = sims.reshape(Q * NCH, CH)
    ans2 = jnp.pad(answers.astype(jnp.int32), (0, NPAD - N)).reshape(NCH, CH)
    return _sc_select(sim2, segmin, v51, ans2)


# speedup vs baseline: 1.2447x; 1.0000x over previous
import dataclasses

import jax
import jax.numpy as jnp
from jax import lax
from jax.experimental import pallas as pl
from jax.experimental.pallas import tpu as pltpu
from jax.experimental.pallas import tpu_sc as plsc

K = 50
NUM_ANSWERS = 1000
Q = 1024
D = 128
N = 100000
BN = 2048
NPAD = 100352

CH = 128
NCH = NPAD // CH
KSEL = 51


def _mm_body(q_ref, kt_ref, o_ref, m_ref):
    i = pl.program_id(0)
    s = jax.lax.dot_general(
        q_ref[...], kt_ref[...], (((1,), (0,)), ((), ())),
        preferred_element_type=jnp.float32,
    )
    col = i * BN + jax.lax.broadcasted_iota(jnp.int32, s.shape, 1)
    s = jnp.where(col < N, s, 3.0)
    o_ref[...] = s
    m_ref[...] = jnp.min(s.reshape(Q, BN // CH, CH), axis=2)[None]


def _sims(qn, knt):
    return pl.pallas_call(
        _mm_body,
        grid=(NPAD // BN,),
        in_specs=[
            pl.BlockSpec((Q, D), lambda i: (0, 0)),
            pl.BlockSpec((D, BN), lambda i: (0, i)),
        ],
        out_specs=[
            pl.BlockSpec((Q, BN), lambda i: (0, i)),
            pl.BlockSpec((1, Q, BN // CH), lambda i: (i, 0, 0)),
        ],
        out_shape=[
            jax.ShapeDtypeStruct((Q, NPAD), jnp.float32),
            jax.ShapeDtypeStruct((NPAD // BN, Q, BN // CH), jnp.float32),
        ],
    )(qn, knt)


def _bisect_body(m_ref, o_ref):
    seg = m_ref[...]

    def step(_, lohi):
        lo, hi = lohi
        mid = (lo + hi) * 0.5
        cnt = jnp.sum((seg <= mid).astype(jnp.float32), axis=1, keepdims=True)
        ge = cnt >= KSEL
        return jnp.where(ge, lo, mid), jnp.where(ge, mid, hi)

    lo0 = jnp.full((Q, 1), -1.5, jnp.float32)
    hi0 = jnp.full((Q, 1), 3.5, jnp.float32)
    _, hi = jax.lax.fori_loop(0, 40, step, (lo0, hi0))
    o_ref[...] = hi


def _v51(segmin):
    return pl.pallas_call(
        _bisect_body,
        out_shape=jax.ShapeDtypeStruct((Q, 1), jnp.float32),
    )(segmin)


NW = 32
QPW = Q // NW
L = 16
CAPCH = 64
IDBUF = CAPCH + L
CANDB = CAPCH * CH + L
BIGF = 1e30
BIGI = 2**31 - 1
PADLAB = 1 << 20


def _sc_body(sim2_hbm, seg_hbm, v51_hbm, ans2_hbm, out_hbm,
             seg16a, seg16b, v51b, idsb, rowb, gb, ab, cval, clab,
             kbuf, cbuf, outb, sem, sem2):
    wid = lax.axis_index("s") * 2 + lax.axis_index("c")
    qbase = wid * QPW
    pltpu.sync_copy(v51_hbm.at[pl.ds(qbase, QPW)], v51b)
    lane = lax.iota(jnp.int32, L)
    pltpu.sync_copy(seg_hbm.at[pl.ds(qbase, L)], seg16a)
    cpj = pltpu.async_copy(seg_hbm.at[pl.ds(qbase + L, L)], seg16b, sem2)

    for j in range(QPW // L):
        if j > 0:
            cpj.wait()
        seg16 = seg16a if j == 0 else seg16b
        def qstep(t, outv):
            q = qbase + j * L + t
            vv = v51b[pl.ds(j * L, L)]
            tsc = lax.reduce_min(jnp.where(lane == t, vv, BIGF), (0,))
            tv = jnp.full((L,), tsc)

            for g in range(IDBUF // L):
                idsb[pl.ds(g * L, L)] = jnp.zeros((L,), jnp.int32)

            def p1(g, off):
                m = seg16[t, pl.ds(g * L, L)] <= tv
                m = jnp.logical_and(m, jnp.full((L,), off < CAPCH))
                plsc.store_compressed(idsb.at[pl.ds(off, L)], g * L + lane,
                                      mask=m)
                return off + lax.reduce_max(
                    plsc.all_reduce_population_count(m), (0,))

            off = lax.fori_loop(0, NCH // L, p1, jnp.int32(0))
            nch = jnp.minimum(off, jnp.int32(CAPCH))

            qrow = jnp.full((L,), q * NCH, jnp.int32)
            for g in range(CAPCH // L):
                rowb[pl.ds(g * L, L)] = idsb[pl.ds(g * L, L)] + qrow

            cp1 = pltpu.async_copy(
                sim2_hbm.at[rowb.at[pl.ds(0, CAPCH)]], gb, sem)
            cp2 = pltpu.async_copy(
                ans2_hbm.at[idsb.at[pl.ds(0, CAPCH)]], ab, sem)
            cp1.wait()
            cp2.wait()

            def p2(r, offc):
                res = offc
                for c in range(0, CH, L):
                    v = gb[r, pl.ds(c, L)]
                    m = v <= tv
                    plsc.store_compressed(cval.at[pl.ds(res, L)], v, mask=m)
                    plsc.store_compressed(clab.at[pl.ds(res, L)],
                                          ab[r, pl.ds(c, L)], mask=m)
                    res = res + lax.reduce_max(
                        plsc.all_reduce_population_count(m), (0,))
                return res

            ccnt = lax.fori_loop(0, nch, p2, jnp.int32(0))
            cval[pl.ds(ccnt, L)] = jnp.full((L,), BIGF, jnp.float32)
            clab[pl.ds(ccnt, L)] = jnp.full((L,), PADLAB, jnp.int32)
            nv = (ccnt + L - 1) // L

            def p3(ga, acc):
                va = cval[pl.ds(ga * L, L)]
                pa = ga * L + lane

                def p3b(gb_, rank):
                    vb = cval[pl.ds(gb_ * L, L)]
                    for k in range(L):
                        ridx = (lane + k) & (L - 1)
                        vr = vb.at[ridx].get(mode="promise_in_bounds")
                        pr = gb_ * L + ridx
                        less = (vr < va) | ((vr == va) & (pr < pa))
                        rank = rank + less.astype(jnp.int32)
                    return rank

                rank = lax.fori_loop(0, nv, p3b, jnp.zeros((L,), jnp.int32))
                kbuf[pl.ds(ga * L, L)] = (rank < K).astype(jnp.int32)
                return acc

            lax.fori_loop(0, nv, p3, jnp.int32(0))

            def p4(ga, mc):
                la = clab[pl.ds(ga * L, L)]
                ka = kbuf[pl.ds(ga * L, L)]

                def p4b(gb_, cnt):
                    lb = clab[pl.ds(gb_ * L, L)]
                    kb = kbuf[pl.ds(gb_ * L, L)]
                    for k in range(L):
                        ridx = (lane + k) & (L - 1)
                        lr = lb.at[ridx].get(mode="promise_in_bounds")
                        kr = kb.at[ridx].get(mode="promise_in_bounds")
                        cnt = cnt + jnp.where(lr == la, kr, 0)
                    return cnt

                cnt = lax.fori_loop(0, nv, p4b, jnp.zeros((L,), jnp.int32))
                cbuf[pl.ds(ga * L, L)] = cnt
                return jnp.maximum(mc, lax.reduce_max(
                    jnp.where(ka > 0, cnt, -1), (0,)))

            maxc = lax.fori_loop(0, nv, p4, jnp.int32(-1))

            def p5(ga, w):
                la = clab[pl.ds(ga * L, L)]
                ka = kbuf[pl.ds(ga * L, L)]
                ca = cbuf[pl.ds(ga * L, L)]
                sel = (ka > 0) & (ca == maxc)
                return jnp.minimum(w, lax.reduce_min(
                    jnp.where(sel, la, BIGI), (0,)))

            win = lax.fori_loop(0, nv, p5, jnp.int32(BIGI))
            return jnp.where(lane == t, win, outv)

        outv = lax.fori_loop(0, L, qstep, jnp.zeros((L,), jnp.int32))
        outb[pl.ds(j * L, L)] = outv

    pltpu.sync_copy(outb, out_hbm.at[pl.ds(qbase, QPW)])


def _sc_params():
    cp = pltpu.CompilerParams()
    if "needs_layout_passes" in pltpu.CompilerParams.__dataclass_fields__:
        cp = dataclasses.replace(cp, needs_layout_passes=False)
    return cp


def _sc_select(sim2, segmin, v51, ans2):
    fn = pl.kernel(
        _sc_body,
        mesh=plsc.VectorSubcoreMesh(core_axis_name="c", subcore_axis_name="s"),
        compiler_params=_sc_params(),
        out_type=jax.ShapeDtypeStruct((Q,), jnp.int32),
        scratch_types=[
            pltpu.VMEM((L, NCH), jnp.float32),
            pltpu.VMEM((L, NCH), jnp.float32),
            pltpu.VMEM((QPW,), jnp.float32),
            pltpu.VMEM((IDBUF,), jnp.int32),
            pltpu.VMEM((IDBUF,), jnp.int32),
            pltpu.VMEM((CAPCH, CH), jnp.float32),
            pltpu.VMEM((CAPCH, CH), jnp.int32),
            pltpu.VMEM((CANDB,), jnp.float32),
            pltpu.VMEM((CANDB,), jnp.int32),
            pltpu.VMEM((CANDB,), jnp.int32),
            pltpu.VMEM((CANDB,), jnp.int32),
            pltpu.VMEM((QPW,), jnp.int32),
            pltpu.SemaphoreType.DMA,
            pltpu.SemaphoreType.DMA,
        ],
    )
    return fn(sim2, segmin, v51, ans2)


def kernel(queries, keys, answers):
    qn = queries / (jnp.linalg.norm(queries, axis=1, keepdims=True) + 1e-8)
    kn = keys / (jnp.linalg.norm(keys, axis=1, keepdims=True) + 1e-8)
    knp = jnp.pad(kn, ((0, NPAD - N), (0, 0)))
    sims, segmin3 = _sims(qn, knp.T)
    segmin = segmin3.transpose(1, 0, 2).reshape(Q, NCH)
    v51 = _v51(segmin).reshape(Q)
    sim2 = sims.reshape(Q * NCH, CH)
    ans2 = jnp.pad(answers.astype(jnp.int32), (0, NPAD - N)).reshape(NCH, CH)
    return _sc_select(sim2, segmin, v51, ans2)


# speedup vs baseline: 1.2491x; 1.0035x over previous
import dataclasses

import jax
import jax.numpy as jnp
from jax import lax
from jax.experimental import pallas as pl
from jax.experimental.pallas import tpu as pltpu
from jax.experimental.pallas import tpu_sc as plsc

K = 50
NUM_ANSWERS = 1000
Q = 1024
D = 128
N = 100000
BN = 2048
NPAD = 100352

CH = 128
NCH = NPAD // CH
KSEL = 51


def _mm_body(q_ref, kt_ref, o_ref, m_ref):
    i = pl.program_id(0)
    s = jax.lax.dot_general(
        q_ref[...], kt_ref[...], (((1,), (0,)), ((), ())),
        preferred_element_type=jnp.float32,
    )
    col = i * BN + jax.lax.broadcasted_iota(jnp.int32, s.shape, 1)
    s = jnp.where(col < N, s, 3.0)
    o_ref[...] = s
    m_ref[...] = jnp.min(s.reshape(Q, BN // CH, CH), axis=2)[None]


def _sims(qn, knt):
    return pl.pallas_call(
        _mm_body,
        grid=(NPAD // BN,),
        in_specs=[
            pl.BlockSpec((Q, D), lambda i: (0, 0)),
            pl.BlockSpec((D, BN), lambda i: (0, i)),
        ],
        out_specs=[
            pl.BlockSpec((Q, BN), lambda i: (0, i)),
            pl.BlockSpec((1, Q, BN // CH), lambda i: (i, 0, 0)),
        ],
        out_shape=[
            jax.ShapeDtypeStruct((Q, NPAD), jnp.float32),
            jax.ShapeDtypeStruct((NPAD // BN, Q, BN // CH), jnp.float32),
        ],
    )(qn, knt)


def _bisect_body(m_ref, o_ref):
    seg = m_ref[...]

    def step(_, lohi):
        lo, hi = lohi
        mid = (lo + hi) * 0.5
        cnt = jnp.sum((seg <= mid).astype(jnp.float32), axis=1, keepdims=True)
        ge = cnt >= KSEL
        return jnp.where(ge, lo, mid), jnp.where(ge, mid, hi)

    lo0 = jnp.full((Q, 1), -1.5, jnp.float32)
    hi0 = jnp.full((Q, 1), 3.5, jnp.float32)
    _, hi = jax.lax.fori_loop(0, 40, step, (lo0, hi0))
    o_ref[...] = hi


def _v51(segmin):
    return pl.pallas_call(
        _bisect_body,
        out_shape=jax.ShapeDtypeStruct((Q, 1), jnp.float32),
    )(segmin)


NW = 32
QPW = Q // NW
L = 16
CAPCH = 64
IDBUF = CAPCH + L
CANDB = CAPCH * CH + L
BIGF = 1e30
BIGI = 2**31 - 1
PADLAB = 1 << 20


def _sc_body(sim2_hbm, seg_hbm, v51_hbm, ans2_hbm, out_hbm,
             seg16a, seg16b, v51b, idsb, rowb, gb, ab, cval, clab,
             kbuf, cbuf, outb, sem, sem2):
    wid = lax.axis_index("s") * 2 + lax.axis_index("c")
    qbase = wid * QPW
    pltpu.sync_copy(v51_hbm.at[pl.ds(qbase, QPW)], v51b)
    lane = lax.iota(jnp.int32, L)
    pltpu.sync_copy(seg_hbm.at[pl.ds(qbase, L)], seg16a)
    cpj = pltpu.async_copy(seg_hbm.at[pl.ds(qbase + L, L)], seg16b, sem2)

    for j in range(QPW // L):
        if j > 0:
            cpj.wait()
        seg16 = seg16a if j == 0 else seg16b
        def qstep(t, outv):
            q = qbase + j * L + t
            vv = v51b[pl.ds(j * L, L)]
            tsc = lax.reduce_min(jnp.where(lane == t, vv, BIGF), (0,))
            tv = jnp.full((L,), tsc)

            for g in range(IDBUF // L):
                idsb[pl.ds(g * L, L)] = jnp.zeros((L,), jnp.int32)

            def p1(g, off):
                m = seg16[t, pl.ds(g * L, L)] <= tv
                m = jnp.logical_and(m, jnp.full((L,), off < CAPCH))
                plsc.store_compressed(idsb.at[pl.ds(off, L)], g * L + lane,
                                      mask=m)
                return off + lax.reduce_max(
                    plsc.all_reduce_population_count(m), (0,))

            off = lax.fori_loop(0, NCH // L, p1, jnp.int32(0))
            nch = jnp.minimum(off, jnp.int32(CAPCH))

            qrow = jnp.full((L,), q * NCH, jnp.int32)
            for g in range(CAPCH // L):
                rowb[pl.ds(g * L, L)] = idsb[pl.ds(g * L, L)] + qrow

            cp1 = pltpu.async_copy(
                sim2_hbm.at[rowb.at[pl.ds(0, CAPCH)]], gb, sem)
            cp2 = pltpu.async_copy(
                ans2_hbm.at[idsb.at[pl.ds(0, CAPCH)]], ab, sem)
            cp1.wait()
            cp2.wait()

            def p2(r, offc):
                res = offc
                for c in range(0, CH, L):
                    v = gb[r, pl.ds(c, L)]
                    m = v <= tv
                    plsc.store_compressed(cval.at[pl.ds(res, L)], v, mask=m)
                    plsc.store_compressed(clab.at[pl.ds(res, L)],
                                          ab[r, pl.ds(c, L)], mask=m)
                    res = res + lax.reduce_max(
                        plsc.all_reduce_population_count(m), (0,))
                return res

            ccnt = lax.fori_loop(0, nch, p2, jnp.int32(0))
            cval[pl.ds(ccnt, L)] = jnp.full((L,), BIGF, jnp.float32)
            clab[pl.ds(ccnt, L)] = jnp.full((L,), PADLAB, jnp.int32)
            nv = (ccnt + L - 1) // L

            def kinit(ga, acc):
                kbuf[pl.ds(ga * L, L)] = (
                    ga * L + lane < ccnt).astype(jnp.int32)
                return acc

            lax.fori_loop(0, nv, kinit, jnp.int32(0))

            def pdrop(it, acc):
                def fmax(ga, mv):
                    va = jnp.where(kbuf[pl.ds(ga * L, L)] > 0,
                                   cval[pl.ds(ga * L, L)], -BIGF)
                    return jnp.maximum(mv, lax.reduce_max(va, (0,)))

                m = lax.fori_loop(0, nv, fmax, jnp.float32(-BIGF))

                def fpos(ga, mp):
                    sel = jnp.logical_and(kbuf[pl.ds(ga * L, L)] > 0,
                                          cval[pl.ds(ga * L, L)] == m)
                    return jnp.maximum(mp, lax.reduce_max(
                        jnp.where(sel, ga * L + lane, -1), (0,)))

                p = lax.fori_loop(0, nv, fpos, jnp.int32(-1))

                def fclr(ga, acc2):
                    pa = ga * L + lane
                    kbuf[pl.ds(ga * L, L)] = jnp.where(
                        pa == p, 0, kbuf[pl.ds(ga * L, L)])
                    return acc2

                lax.fori_loop(0, nv, fclr, jnp.int32(0))
                return acc

            lax.fori_loop(0, ccnt - K, pdrop, jnp.int32(0))

            def p4(ga, mc):
                la = clab[pl.ds(ga * L, L)]
                ka = kbuf[pl.ds(ga * L, L)]

                def p4b(gb_, cnt):
                    lb = clab[pl.ds(gb_ * L, L)]
                    kb = kbuf[pl.ds(gb_ * L, L)]
                    for k in range(L):
                        ridx = (lane + k) & (L - 1)
                        lr = lb.at[ridx].get(mode="promise_in_bounds")
                        kr = kb.at[ridx].get(mode="promise_in_bounds")
                        cnt = cnt + jnp.where(lr == la, kr, 0)
                    return cnt

                cnt = lax.fori_loop(0, nv, p4b, jnp.zeros((L,), jnp.int32))
                cbuf[pl.ds(ga * L, L)] = cnt
                return jnp.maximum(mc, lax.reduce_max(
                    jnp.where(ka > 0, cnt, -1), (0,)))

            maxc = lax.fori_loop(0, nv, p4, jnp.int32(-1))

            def p5(ga, w):
                la = clab[pl.ds(ga * L, L)]
                ka = kbuf[pl.ds(ga * L, L)]
                ca = cbuf[pl.ds(ga * L, L)]
                sel = (ka > 0) & (ca == maxc)
                return jnp.minimum(w, lax.reduce_min(
                    jnp.where(sel, la, BIGI), (0,)))

            win = lax.fori_loop(0, nv, p5, jnp.int32(BIGI))
            return jnp.where(lane == t, win, outv)

        outv = lax.fori_loop(0, L, qstep, jnp.zeros((L,), jnp.int32))
        outb[pl.ds(j * L, L)] = outv

    pltpu.sync_copy(outb, out_hbm.at[pl.ds(qbase, QPW)])


def _sc_params():
    cp = pltpu.CompilerParams()
    if "needs_layout_passes" in pltpu.CompilerParams.__dataclass_fields__:
        cp = dataclasses.replace(cp, needs_layout_passes=False)
    return cp


def _sc_select(sim2, segmin, v51, ans2):
    fn = pl.kernel(
        _sc_body,
        mesh=plsc.VectorSubcoreMesh(core_axis_name="c", subcore_axis_name="s"),
        compiler_params=_sc_params(),
        out_type=jax.ShapeDtypeStruct((Q,), jnp.int32),
        scratch_types=[
            pltpu.VMEM((L, NCH), jnp.float32),
            pltpu.VMEM((L, NCH), jnp.float32),
            pltpu.VMEM((QPW,), jnp.float32),
            pltpu.VMEM((IDBUF,), jnp.int32),
            pltpu.VMEM((IDBUF,), jnp.int32),
            pltpu.VMEM((CAPCH, CH), jnp.float32),
            pltpu.VMEM((CAPCH, CH), jnp.int32),
            pltpu.VMEM((CANDB,), jnp.float32),
            pltpu.VMEM((CANDB,), jnp.int32),
            pltpu.VMEM((CANDB,), jnp.int32),
            pltpu.VMEM((CANDB,), jnp.int32),
            pltpu.VMEM((QPW,), jnp.int32),
            pltpu.SemaphoreType.DMA,
            pltpu.SemaphoreType.DMA,
        ],
    )
    return fn(sim2, segmin, v51, ans2)


def kernel(queries, keys, answers):
    qn = queries / (jnp.linalg.norm(queries, axis=1, keepdims=True) + 1e-8)
    kn = keys / (jnp.linalg.norm(keys, axis=1, keepdims=True) + 1e-8)
    knp = jnp.pad(kn, ((0, NPAD - N), (0, 0)))
    sims, segmin3 = _sims(qn, knp.T)
    segmin = segmin3.transpose(1, 0, 2).reshape(Q, NCH)
    v51 = _v51(segmin).reshape(Q)
    sim2 = sims.reshape(Q * NCH, CH)
    ans2 = jnp.pad(answers.astype(jnp.int32), (0, NPAD - N)).reshape(NCH, CH)
    return _sc_select(sim2, segmin, v51, ans2)
